# baseline (speedup 1.0000x reference)
import dataclasses

import jax
import jax.numpy as jnp
from jax import lax
from jax.experimental import pallas as pl
from jax.experimental.pallas import tpu as pltpu
from jax.experimental.pallas import tpu_sc as plsc

NUM_CORES = 2
NUM_SUBCORES = 16
NUM_TILES = NUM_CORES * NUM_SUBCORES
LANES = 16

NUM_ENT = 1000000
BATCH = 16384
DIM = 64

C = 256
NB_FULL = 3906
TAIL_BLK = NB_FULL
TAIL_WIN0 = NUM_ENT - 128
TAIL_OWNER = TAIL_BLK % NUM_TILES
KITER = 124

RING = 64
SCAT_CAP = 32
DUMMY_B = BATCH
STAGE_ROWS = BATCH + 8
PAD_IDX = 1 << 26

ROWS_PER_TILE = BATCH // NUM_TILES
CHUNK = 128
NCHUNK = ROWS_PER_TILE // CHUNK
DIM_VREGS = DIM // LANES


def _wid():
    return lax.axis_index("s") * NUM_CORES + lax.axis_index("c")


def _p1_body(entT, ent_tailT, h_hbm, t_hbm, hs, ts,
             hidx, tidx, wlh, wlt, bbuf0, bbuf1, hitc, hitb,
             rh0, rh1, rt0, rt1, bh0, bh1, bt0, bt1, cnt,
             sem_b0, sem_b1, sem_h0, sem_h1, sem_t0, sem_t1):
    wid = _wid()
    iota16 = lax.iota(jnp.int32, LANES)
    zeros16 = jnp.zeros((LANES,), jnp.int32)
    dummy16 = jnp.full((LANES,), DUMMY_B, jnp.int32)

    NWL_H, NWL_T, WR, RD, FILL_H, FILL_T, PAR_H, PAR_T = range(8)

    tables = (
        dict(wl=wlh, idxb=hidx, stage=hs, rows=(rh0, rh1), bidx=(bh0, bh1),
             sems=(sem_h0, sem_h1), NWL=NWL_H, FILL=FILL_H, PAR=PAR_H),
        dict(wl=wlt, idxb=tidx, stage=ts, rows=(rt0, rt1), bidx=(bt0, bt1),
             sems=(sem_t0, sem_t1), NWL=NWL_T, FILL=FILL_T, PAR=PAR_T),
    )

    pltpu.sync_copy(h_hbm, hidx.at[pl.ds(0, BATCH)])
    pltpu.sync_copy(t_hbm, tidx.at[pl.ds(0, BATCH)])
    hidx[pl.ds(BATCH, LANES)] = jnp.full((LANES,), PAD_IDX, jnp.int32)
    tidx[pl.ds(BATCH, LANES)] = jnp.full((LANES,), PAD_IDX, jnp.int32)

    for tb in tables:
        wl, idxb = tb["wl"], tb["idxb"]
        cnt[tb["NWL"]] = 0

        @pl.loop(0, BATCH // LANES)
        def _(i):
            v = idxb[pl.ds(i * LANES, LANES)]
            m = (lax.shift_right_logical(v, 8) & (NUM_TILES - 1)) == wid

            @pl.when(jnp.any(m))
            def _():
                off = cnt[tb["NWL"]]
                plsc.store_compressed(wl.at[pl.ds(off, LANES)],
                                      i * LANES + iota16, mask=m)
                cnt[tb["NWL"]] = off + jnp.max(
                    plsc.all_reduce_population_count(m))

        wl[pl.ds(cnt[tb["NWL"]], LANES)] = dummy16

    def scat_wait(tb, p):
        pltpu.make_async_copy(tb["stage"].at[pl.ds(0, SCAT_CAP), :],
                              tb["rows"][p], tb["sems"][p]).wait()

    for tb in tables:
        for p in (0, 1):
            tb["bidx"][p][pl.ds(0, LANES)] = dummy16
            tb["bidx"][p][pl.ds(LANES, LANES)] = dummy16
            pltpu.async_copy(tb["rows"][p], tb["stage"].at[tb["bidx"][p]],
                             tb["sems"][p])
        scat_wait(tb, 0)
        cnt[tb["FILL"]] = 0
        cnt[tb["PAR"]] = 0
    cnt[WR] = 0
    cnt[RD] = 0

    def flush(tb):
        for p in (0, 1):
            @pl.when(cnt[tb["PAR"]] == p)
            def _():
                pltpu.async_copy(tb["rows"][p],
                                 tb["stage"].at[tb["bidx"][p]], tb["sems"][p])
        cnt[tb["PAR"]] = 1 - cnt[tb["PAR"]]
        for p in (0, 1):
            @pl.when(cnt[tb["PAR"]] == p)
            def _():
                scat_wait(tb, p)
                tb["bidx"][p][pl.ds(0, LANES)] = dummy16
                tb["bidx"][p][pl.ds(LANES, LANES)] = dummy16
        cnt[tb["FILL"]] = 0

    def drain_one(tb, bbuf):
        rdm = cnt[RD] & (RING - 1)
        colv = hitc[pl.ds(rdm, LANES)]
        bv = hitb[pl.ds(rdm, LANES)]
        fill = cnt[tb["FILL"]]
        slots = fill + iota16
        for p in (0, 1):
            @pl.when(cnt[tb["PAR"]] == p)
            def _():
                rows = tb["rows"][p]

                @pl.loop(0, DIM // LANES)
                def _(dq):
                    for du in range(LANES):
                        d = dq * LANES + du
                        vals = plsc.load_gather(bbuf, [zeros16 + d, colv])
                        plsc.store_scatter(rows, [slots, zeros16 + d], vals)
                tb["bidx"][p][pl.ds(fill, LANES)] = bv
        cnt[RD] = cnt[RD] + LANES
        cnt[tb["FILL"]] = fill + LANES

        @pl.when(cnt[tb["FILL"]] == SCAT_CAP)
        def _():
            flush(tb)

    def scan_block(tb, bbuf, blk, c0):
        wl, idxb = tb["wl"], tb["idxb"]
        nv = (cnt[tb["NWL"]] + LANES - 1) >> 4

        @pl.loop(0, nv)
        def _(i):
            bv = wl[pl.ds(i * LANES, LANES)]
            iv = plsc.load_gather(idxb, [bv])
            m = lax.shift_right_logical(iv, 8) == blk

            @pl.when(jnp.any(m))
            def _():
                colv = iv - c0
                rank = plsc.cumsum(m.astype(jnp.int32))
                wr = cnt[WR]
                wrm = wr & (RING - 1)
                slots = wrm + rank - 1
                plsc.store_scatter(hitc, [slots], colv, mask=m)
                plsc.store_scatter(hitb, [slots], bv, mask=m)
                p = jnp.max(plsc.all_reduce_population_count(m))
                cnt[WR] = wr + p

                @pl.when(wrm + p > RING)
                def _():
                    hitc[pl.ds(0, LANES)] = hitc[pl.ds(RING, LANES)]
                    hitb[pl.ds(0, LANES)] = hitb[pl.ds(RING, LANES)]

            @pl.when(cnt[WR] - cnt[RD] >= LANES)
            def _():
                drain_one(tb, bbuf)

        @pl.when(cnt[WR] - cnt[RD] > 0)
        def _():
            wrm = cnt[WR] & (RING - 1)
            hitc[pl.ds(wrm, LANES)] = zeros16
            hitb[pl.ds(wrm, LANES)] = dummy16

            @pl.when(wrm + LANES > RING)
            def _():
                hitc[pl.ds(0, LANES)] = hitc[pl.ds(RING, LANES)]
                hitb[pl.ds(0, LANES)] = hitb[pl.ds(RING, LANES)]
            cnt[WR] = cnt[RD] + LANES
            drain_one(tb, bbuf)

    def issue_blk(k, bbuf, sem):
        blk = jnp.minimum(wid + NUM_TILES * k, NB_FULL - 1)
        pltpu.async_copy(entT.at[:, pl.ds(blk * C, C)], bbuf, sem)

    def wait_blk(bbuf, sem):
        pltpu.make_async_copy(entT.at[:, pl.ds(0, C)], bbuf, sem).wait()

    def process(k, bbuf):
        blk = wid + NUM_TILES * k

        @pl.when(blk < NB_FULL)
        def _():
            for tb in tables:
                scan_block(tb, bbuf, blk, blk * C)

    issue_blk(0, bbuf0, sem_b0)
    issue_blk(1, bbuf1, sem_b1)

    @pl.loop(0, KITER // 2)
    def _(pi):
        k0 = 2 * pi
        wait_blk(bbuf0, sem_b0)
        process(k0, bbuf0)

        @pl.when(k0 + 2 < KITER)
        def _():
            issue_blk(k0 + 2, bbuf0, sem_b0)
        wait_blk(bbuf1, sem_b1)
        process(k0 + 1, bbuf1)

        @pl.when(k0 + 3 < KITER)
        def _():
            issue_blk(k0 + 3, bbuf1, sem_b1)

    @pl.when(wid == TAIL_OWNER)
    def _():
        pltpu.sync_copy(ent_tailT, bbuf0.at[:, pl.ds(0, 128)])
        for tb in tables:
            scan_block(tb, bbuf0, TAIL_BLK, TAIL_WIN0)

    for tb in tables:
        flush(tb)
        for p in (0, 1):
            @pl.when(cnt[tb["PAR"]] == p)
            def _():
                scat_wait(tb, 1 - p)


def _p2_body(hs, ts, rel_hbm, r_hbm, out_hbm,
             ridx, h_bufs, r_bufs, t_bufs, q, out_v, sem0, sem1):
    wid = _wid()
    base = wid * ROWS_PER_TILE
    pltpu.sync_copy(r_hbm.at[pl.ds(base, ROWS_PER_TILE)], ridx)

    sems = (sem0, sem1)

    def issue(c):
        par = c % 2
        row0 = base + c * CHUNK
        sl = pl.ds(c * CHUNK, CHUNK)
        return [
            pltpu.async_copy(hs.at[pl.ds(row0, CHUNK), :], h_bufs.at[par], sems[par]),
            pltpu.async_copy(ts.at[pl.ds(row0, CHUNK), :], t_bufs.at[par], sems[par]),
            pltpu.async_copy(rel_hbm.at[ridx.at[sl]], r_bufs.at[par], sems[par]),
        ]

    pending = issue(0)
    for c in range(NCHUNK):
        current = pending
        if c + 1 < NCHUNK:
            pending = issue(c + 1)
        for cp in current:
            cp.wait()
        par = c % 2
        hb, rb, tb = h_bufs.at[par], r_bufs.at[par], t_bufs.at[par]

        @pl.loop(0, CHUNK)
        def _(i):
            acc = (hb[i, pl.ds(0, LANES)]
                   * rb[i, pl.ds(0, LANES)]
                   * tb[i, pl.ds(0, LANES)])
            for d in range(1, DIM_VREGS):
                acc = acc + (hb[i, pl.ds(d * LANES, LANES)]
                             * rb[i, pl.ds(d * LANES, LANES)]
                             * tb[i, pl.ds(d * LANES, LANES)])
            q[c * CHUNK + i, :] = acc

    lanes_iota = lax.iota(jnp.int32, LANES)

    @pl.loop(0, ROWS_PER_TILE, step=LANES)
    def _(i0):
        rows16 = i0 + lanes_iota
        acc = plsc.load_gather(q, [rows16, jnp.zeros((LANES,), jnp.int32)])
        for l in range(1, LANES):
            acc = acc + plsc.load_gather(
                q, [rows16, jnp.full((LANES,), l, jnp.int32)])
        out_v[pl.ds(i0, LANES)] = acc

    pltpu.sync_copy(out_v, out_hbm.at[pl.ds(base, ROWS_PER_TILE)])


def _compiler_params(tc_tiling):
    cp = pltpu.CompilerParams()
    fields = pltpu.CompilerParams.__dataclass_fields__
    if "needs_layout_passes" in fields:
        cp = dataclasses.replace(cp, needs_layout_passes=False)
    if "use_tc_tiling_on_sc" in fields:
        cp = dataclasses.replace(cp, use_tc_tiling_on_sc=tc_tiling)
    return cp


@jax.jit
def kernel(entity_embeddings, relation_embeddings, h, r, t):
    entT = jnp.swapaxes(entity_embeddings, 0, 1)
    ent_tailT = jax.lax.slice(entT, (0, TAIL_WIN0), (DIM, NUM_ENT))
    mesh = plsc.VectorSubcoreMesh(core_axis_name="c", subcore_axis_name="s")

    p1 = pl.kernel(
        _p1_body,
        out_type=(jax.ShapeDtypeStruct((STAGE_ROWS, 128), jnp.float32),
                  jax.ShapeDtypeStruct((STAGE_ROWS, 128), jnp.float32)),
        mesh=mesh,
        scratch_types=[
            pltpu.VMEM((BATCH + LANES,), jnp.int32),
            pltpu.VMEM((BATCH + LANES,), jnp.int32),
            pltpu.VMEM((BATCH + LANES,), jnp.int32),
            pltpu.VMEM((BATCH + LANES,), jnp.int32),
            pltpu.VMEM((DIM, C), jnp.float32),
            pltpu.VMEM((DIM, C), jnp.float32),
            pltpu.VMEM((RING + LANES,), jnp.int32),
            pltpu.VMEM((RING + LANES,), jnp.int32),
            pltpu.VMEM((SCAT_CAP, 128), jnp.float32),
            pltpu.VMEM((SCAT_CAP, 128), jnp.float32),
            pltpu.VMEM((SCAT_CAP, 128), jnp.float32),
            pltpu.VMEM((SCAT_CAP, 128), jnp.float32),
            pltpu.VMEM((SCAT_CAP,), jnp.int32),
            pltpu.VMEM((SCAT_CAP,), jnp.int32),
            pltpu.VMEM((SCAT_CAP,), jnp.int32),
            pltpu.VMEM((SCAT_CAP,), jnp.int32),
            pltpu.SMEM((16,), jnp.int32),
            pltpu.SemaphoreType.DMA,
            pltpu.SemaphoreType.DMA,
            pltpu.SemaphoreType.DMA,
            pltpu.SemaphoreType.DMA,
            pltpu.SemaphoreType.DMA,
            pltpu.SemaphoreType.DMA,
        ],
        compiler_params=_compiler_params(True),
    )
    h_stage, t_stage = p1(entT, ent_tailT,
                          h.astype(jnp.int32), t.astype(jnp.int32))

    p2 = pl.kernel(
        _p2_body,
        out_type=jax.ShapeDtypeStruct((BATCH,), jnp.float32),
        mesh=mesh,
        scratch_types=[
            pltpu.VMEM((ROWS_PER_TILE,), jnp.int32),
            pltpu.VMEM((2, CHUNK, 128), jnp.float32),
            pltpu.VMEM((2, CHUNK, DIM), jnp.float32),
            pltpu.VMEM((2, CHUNK, 128), jnp.float32),
            pltpu.VMEM((ROWS_PER_TILE, LANES), jnp.float32),
            pltpu.VMEM((ROWS_PER_TILE,), jnp.float32),
            pltpu.SemaphoreType.DMA,
            pltpu.SemaphoreType.DMA,
        ],
        compiler_params=_compiler_params(False),
    )
    return p2(h_stage, t_stage, relation_embeddings, r.astype(jnp.int32))
